# Initial kernel scaffold; baseline (speedup 1.0000x reference)
import jax
import jax.numpy as jnp
from jax.experimental import pallas as pl


def kernel(adj, weight, features, W_feat, b_feat, W_gnn0, W_gnn1, centroids, W_out, b_out):
    raise NotImplementedError("write your pallas kernel here")


# speedup vs baseline: 1.9486x; 1.9486x over previous
import functools

import jax
import jax.numpy as jnp
from jax import lax
from jax.experimental import pallas as pl
from jax.experimental.pallas import tpu as pltpu
from jax.experimental.pallas import tpu_sc as plsc

N = 10000
NB = 32
DIN = 128
D = 64
K = 100
C = 40

NW = 32
NPAD = 10240
NODES_PER_W = NPAD // NW
CH_NODES = 4
CH_EDGES = CH_NODES * NB
N_CHUNKS = NODES_PER_W // CH_NODES
RB = 1024

_PREC = jax.lax.Precision.HIGHEST


def _proj_body(f_ref, wf_ref, bf_ref, wg_ref, o_ref):
    x = jnp.dot(f_ref[...], wf_ref[...], preferred_element_type=jnp.float32,
                precision=_PREC)
    x = jnp.maximum(x + bf_ref[...], 0.0)
    o_ref[...] = jnp.dot(x, wg_ref[...], preferred_element_type=jnp.float32,
                         precision=_PREC)


def _mm_body(x_ref, w_ref, o_ref):
    o_ref[...] = jnp.dot(x_ref[...], w_ref[...],
                         preferred_element_type=jnp.float32, precision=_PREC)


def _head_body(x_ref, ct_ref, wo_ref, bo_ref, o_ref):
    x = x_ref[...]
    ct = ct_ref[...]
    xsq = jnp.sum(x * x, axis=1, keepdims=True)
    csq = jnp.sum(ct * ct, axis=0, keepdims=True)
    cross = jnp.dot(x, ct, preferred_element_type=jnp.float32, precision=_PREC)
    sq = xsq + csq - 2.0 * cross
    sim = jnp.sqrt(jnp.maximum(sq, 1e-12))
    logit = jnp.dot(sim, wo_ref[...], preferred_element_type=jnp.float32,
                    precision=_PREC) + bo_ref[...]
    m = jnp.max(logit, axis=1, keepdims=True)
    lse = jnp.log(jnp.sum(jnp.exp(logit - m), axis=1, keepdims=True)) + m
    o_ref[...] = logit - lse


def _sc_agg_body(msg_hbm, idx_hbm, w_hbm, out_hbm, idxbuf, wbuf, rows, outbuf, sem):
    wid = lax.axis_index("s") * 2 + lax.axis_index("c")
    node0 = wid * NODES_PER_W
    edge0 = node0 * NB

    def chunk_body(c, carry):
        ebase = edge0 + c * CH_EDGES
        nbase = node0 + c * CH_NODES
        pltpu.sync_copy(idx_hbm.at[pl.ds(ebase, CH_EDGES)], idxbuf)
        pltpu.sync_copy(w_hbm.at[pl.ds(ebase, CH_EDGES)], wbuf)
        pltpu.async_copy(msg_hbm.at[idxbuf], rows, sem).wait()
        for n in range(CH_NODES):
            acc = [jnp.zeros((16,), jnp.float32) for _ in range(4)]
            for g in range(NB // 16):
                wvec = wbuf[pl.ds((n * NB + g * 16), 16)]
                for e in range(16):
                    eidx = n * NB + g * 16 + e
                    wv = wvec[e]
                    for k in range(4):
                        acc[k] = acc[k] + wv * rows[eidx, pl.ds(k * 16, 16)]
            for k in range(4):
                outbuf[n, pl.ds(k * 16, 16)] = jnp.maximum(acc[k], 0.0)
        pltpu.sync_copy(outbuf, out_hbm.at[pl.ds(nbase, CH_NODES)])
        return carry

    lax.fori_loop(0, N_CHUNKS, chunk_body, 0)


def _make_sc_agg():
    return pl.kernel(
        _sc_agg_body,
        out_type=jax.ShapeDtypeStruct((NPAD, D), jnp.float32),
        mesh=plsc.VectorSubcoreMesh(core_axis_name="c", subcore_axis_name="s"),
        compiler_params=pltpu.CompilerParams(use_tc_tiling_on_sc=False),
        scratch_types=[
            pltpu.VMEM((CH_EDGES,), jnp.int32),
            pltpu.VMEM((CH_EDGES,), jnp.float32),
            pltpu.VMEM((CH_EDGES, D), jnp.float32),
            pltpu.VMEM((CH_NODES, D), jnp.float32),
            pltpu.SemaphoreType.DMA,
        ],
    )


def _proj(f, W_feat, b_feat, W_gnn0):
    return pl.pallas_call(
        _proj_body,
        grid=(NPAD // RB,),
        in_specs=[
            pl.BlockSpec((RB, DIN), lambda i: (i, 0)),
            pl.BlockSpec((DIN, D), lambda i: (0, 0)),
            pl.BlockSpec((1, D), lambda i: (0, 0)),
            pl.BlockSpec((D, D), lambda i: (0, 0)),
        ],
        out_specs=pl.BlockSpec((RB, D), lambda i: (i, 0)),
        out_shape=jax.ShapeDtypeStruct((NPAD, D), jnp.float32),
    )(f, W_feat, b_feat, W_gnn0)


def _mm(x, W):
    return pl.pallas_call(
        _mm_body,
        grid=(NPAD // RB,),
        in_specs=[
            pl.BlockSpec((RB, D), lambda i: (i, 0)),
            pl.BlockSpec((D, D), lambda i: (0, 0)),
        ],
        out_specs=pl.BlockSpec((RB, D), lambda i: (i, 0)),
        out_shape=jax.ShapeDtypeStruct((NPAD, D), jnp.float32),
    )(x, W)


def _head(x, ct, W_out, b_out):
    return pl.pallas_call(
        _head_body,
        grid=(NPAD // RB,),
        in_specs=[
            pl.BlockSpec((RB, D), lambda i: (i, 0)),
            pl.BlockSpec((D, K), lambda i: (0, 0)),
            pl.BlockSpec((K, C), lambda i: (0, 0)),
            pl.BlockSpec((1, C), lambda i: (0, 0)),
        ],
        out_specs=pl.BlockSpec((RB, C), lambda i: (i, 0)),
        out_shape=jax.ShapeDtypeStruct((NPAD, C), jnp.float32),
    )(x, ct, W_out, b_out)


def kernel(adj, weight, features, W_feat, b_feat, W_gnn0, W_gnn1, centroids,
           W_out, b_out):
    pad = NPAD - N
    a = jnp.pad(adj[0].astype(jnp.int32), ((0, pad), (0, 0)))
    w = jnp.pad(weight[0].astype(jnp.float32), ((0, pad), (0, 0)))
    f = jnp.pad(features[0], ((0, pad), (0, 0)))
    idx_flat = a.reshape(-1)
    w_flat = w.reshape(-1)

    sc_agg = _make_sc_agg()
    msg0 = _proj(f, W_feat, b_feat.reshape(1, D), W_gnn0)
    x1 = sc_agg(msg0, idx_flat, w_flat)
    msg1 = _mm(x1, W_gnn1)
    x2 = sc_agg(msg1, idx_flat, w_flat)
    out = _head(x2, centroids.T, W_out, b_out.reshape(1, C))
    return out[:N]


# speedup vs baseline: 2.7622x; 1.4175x over previous
import functools

import jax
import jax.numpy as jnp
from jax import lax
from jax.experimental import pallas as pl
from jax.experimental.pallas import tpu as pltpu
from jax.experimental.pallas import tpu_sc as plsc

N = 10000
NB = 32
DIN = 128
D = 64
K = 100
C = 40

NW = 32
NPAD = 10240
NODES_PER_W = NPAD // NW
CH_NODES = 4
CH_EDGES = CH_NODES * NB
N_CHUNKS = NODES_PER_W // CH_NODES
RB = 1024

_PREC = jax.lax.Precision.HIGHEST


def _proj_body(f_ref, wf_ref, bf_ref, wg_ref, o_ref):
    x = jnp.dot(f_ref[...], wf_ref[...], preferred_element_type=jnp.float32,
                precision=_PREC)
    x = jnp.maximum(x + bf_ref[...], 0.0)
    o_ref[...] = jnp.dot(x, wg_ref[...], preferred_element_type=jnp.float32,
                         precision=_PREC)


def _mm_body(x_ref, w_ref, o_ref):
    o_ref[...] = jnp.dot(x_ref[...], w_ref[...],
                         preferred_element_type=jnp.float32, precision=_PREC)


def _head_body(x_ref, ct_ref, wo_ref, bo_ref, o_ref):
    x = x_ref[...]
    ct = ct_ref[...]
    xsq = jnp.sum(x * x, axis=1, keepdims=True)
    csq = jnp.sum(ct * ct, axis=0, keepdims=True)
    cross = jnp.dot(x, ct, preferred_element_type=jnp.float32, precision=_PREC)
    sq = xsq + csq - 2.0 * cross
    sim = jnp.sqrt(jnp.maximum(sq, 1e-12))
    logit = jnp.dot(sim, wo_ref[...], preferred_element_type=jnp.float32,
                    precision=_PREC) + bo_ref[...]
    m = jnp.max(logit, axis=1, keepdims=True)
    lse = jnp.log(jnp.sum(jnp.exp(logit - m), axis=1, keepdims=True)) + m
    o_ref[...] = logit - lse


def _sc_agg_body(msg_hbm, idx_hbm, w_hbm, out_hbm, idx_all, w_all, rows, outbuf,
                 sem0, sem1, osem0, osem1):
    wid = lax.axis_index("s") * 2 + lax.axis_index("c")
    node0 = wid * NODES_PER_W
    row0 = wid * N_CHUNKS

    pltpu.sync_copy(idx_hbm.at[pl.ds(row0, N_CHUNKS)], idx_all)
    pltpu.sync_copy(w_hbm.at[pl.ds(row0, N_CHUNKS)], w_all)

    sems = (sem0, sem1)
    osems = (osem0, osem1)

    def gather_start(c, b):
        return pltpu.async_copy(msg_hbm.at[idx_all.at[c]], rows.at[b], sems[b])

    gather_start(0, 0)

    def pair_body(i, carry):
        for b in range(2):
            c = i * 2 + b
            nxt = c + 1

            @pl.when(nxt < N_CHUNKS)
            def _():
                gather_start(nxt, 1 - b)

            pltpu.make_async_copy(msg_hbm.at[idx_all.at[c]], rows.at[b],
                                  sems[b]).wait()
            @pl.when(c >= 2)
            def _():
                pltpu.make_async_copy(
                    outbuf.at[b], out_hbm.at[pl.ds(0, CH_NODES)], osems[b]
                ).wait()
            for n in range(CH_NODES):
                acc = [jnp.zeros((16,), jnp.float32) for _ in range(4)]
                for g in range(NB // 16):
                    wvec = w_all[c, pl.ds(n * NB + g * 16, 16)]
                    for e in range(16):
                        eidx = n * NB + g * 16 + e
                        wv = wvec[e]
                        for k in range(4):
                            acc[k] = acc[k] + wv * rows[b, eidx, pl.ds(k * 16, 16)]
                for k in range(4):
                    outbuf[b, n, pl.ds(k * 16, 16)] = jnp.maximum(acc[k], 0.0)
            nbase = node0 + c * CH_NODES
            pltpu.async_copy(outbuf.at[b], out_hbm.at[pl.ds(nbase, CH_NODES)],
                             osems[b])
        return carry

    lax.fori_loop(0, N_CHUNKS // 2, pair_body, 0)
    for b in range(2):
        pltpu.make_async_copy(outbuf.at[b], out_hbm.at[pl.ds(0, CH_NODES)],
                              osems[b]).wait()


def _make_sc_agg():
    return pl.kernel(
        _sc_agg_body,
        out_type=jax.ShapeDtypeStruct((NPAD, D), jnp.float32),
        mesh=plsc.VectorSubcoreMesh(core_axis_name="c", subcore_axis_name="s"),
        compiler_params=pltpu.CompilerParams(use_tc_tiling_on_sc=False),
        scratch_types=[
            pltpu.VMEM((N_CHUNKS, CH_EDGES), jnp.int32),
            pltpu.VMEM((N_CHUNKS, CH_EDGES), jnp.float32),
            pltpu.VMEM((2, CH_EDGES, D), jnp.float32),
            pltpu.VMEM((2, CH_NODES, D), jnp.float32),
            pltpu.SemaphoreType.DMA,
            pltpu.SemaphoreType.DMA,
            pltpu.SemaphoreType.DMA,
            pltpu.SemaphoreType.DMA,
        ],
    )


def _proj(f, W_feat, b_feat, W_gnn0):
    return pl.pallas_call(
        _proj_body,
        grid=(NPAD // RB,),
        in_specs=[
            pl.BlockSpec((RB, DIN), lambda i: (i, 0)),
            pl.BlockSpec((DIN, D), lambda i: (0, 0)),
            pl.BlockSpec((1, D), lambda i: (0, 0)),
            pl.BlockSpec((D, D), lambda i: (0, 0)),
        ],
        out_specs=pl.BlockSpec((RB, D), lambda i: (i, 0)),
        out_shape=jax.ShapeDtypeStruct((NPAD, D), jnp.float32),
    )(f, W_feat, b_feat, W_gnn0)


def _mm(x, W):
    return pl.pallas_call(
        _mm_body,
        grid=(NPAD // RB,),
        in_specs=[
            pl.BlockSpec((RB, D), lambda i: (i, 0)),
            pl.BlockSpec((D, D), lambda i: (0, 0)),
        ],
        out_specs=pl.BlockSpec((RB, D), lambda i: (i, 0)),
        out_shape=jax.ShapeDtypeStruct((NPAD, D), jnp.float32),
    )(x, W)


def _head(x, ct, W_out, b_out):
    return pl.pallas_call(
        _head_body,
        grid=(NPAD // RB,),
        in_specs=[
            pl.BlockSpec((RB, D), lambda i: (i, 0)),
            pl.BlockSpec((D, K), lambda i: (0, 0)),
            pl.BlockSpec((K, C), lambda i: (0, 0)),
            pl.BlockSpec((1, C), lambda i: (0, 0)),
        ],
        out_specs=pl.BlockSpec((RB, C), lambda i: (i, 0)),
        out_shape=jax.ShapeDtypeStruct((NPAD, C), jnp.float32),
    )(x, ct, W_out, b_out)


def kernel(adj, weight, features, W_feat, b_feat, W_gnn0, W_gnn1, centroids,
           W_out, b_out):
    pad = NPAD - N
    a = jnp.pad(adj[0].astype(jnp.int32), ((0, pad), (0, 0)))
    w = jnp.pad(weight[0].astype(jnp.float32), ((0, pad), (0, 0)))
    f = jnp.pad(features[0], ((0, pad), (0, 0)))
    idx_flat = a.reshape(-1, CH_EDGES)
    w_flat = w.reshape(-1, CH_EDGES)

    sc_agg = _make_sc_agg()
    msg0 = _proj(f, W_feat, b_feat.reshape(1, D), W_gnn0)
    x1 = sc_agg(msg0, idx_flat, w_flat)
    msg1 = _mm(x1, W_gnn1)
    x2 = sc_agg(msg1, idx_flat, w_flat)
    out = _head(x2, centroids.T, W_out, b_out.reshape(1, C))
    return out[:N]


# speedup vs baseline: 2.8402x; 1.0283x over previous
import jax
import jax.numpy as jnp
from jax import lax
from jax.experimental import pallas as pl
from jax.experimental.pallas import tpu as pltpu
from jax.experimental.pallas import tpu_sc as plsc

N = 10000
NB = 32
DIN = 128
D = 64
K = 100
C = 40

NW = 32
NPAD = 10240
NODES_PER_W = NPAD // NW
ROWS_PER_W = NODES_PER_W * NB // 128
BC_NODES = 8
BC_GATHERS = BC_NODES * NB // 128
N_BIG = NODES_PER_W // BC_NODES

_PREC = jax.lax.Precision.HIGHEST
_DN = (((1,), (1,)), ((), ()))


def _proj_body(f_ref, wf_ref, bf_ref, wg_ref, o_ref):
    x = jnp.dot(f_ref[...], wf_ref[...], preferred_element_type=jnp.float32,
                precision=_PREC)
    x = jnp.maximum(x + bf_ref[...], 0.0)
    o_ref[...] = jnp.dot(x, wg_ref[...], preferred_element_type=jnp.float32,
                         precision=_PREC)


def _mm_body(x_ref, w_ref, o_ref):
    o_ref[...] = jnp.dot(x_ref[...], w_ref[...],
                         preferred_element_type=jnp.float32, precision=_PREC)


def _head_body(x_ref, c_ref, wo_ref, bo_ref, o_ref):
    x = x_ref[...]
    cen = c_ref[...]
    xsq = jnp.sum(x * x, axis=1, keepdims=True)
    csq = jnp.sum(cen * cen, axis=1)[None, :]
    cross = lax.dot_general(x, cen, _DN, precision=_PREC,
                            preferred_element_type=jnp.float32)
    sq = xsq + csq - 2.0 * cross
    sim = jnp.sqrt(jnp.maximum(sq, 1e-12))
    logit = jnp.dot(sim, wo_ref[...], preferred_element_type=jnp.float32,
                    precision=_PREC) + bo_ref[...]
    m = jnp.max(logit, axis=1, keepdims=True)
    lse = jnp.log(jnp.sum(jnp.exp(logit - m), axis=1, keepdims=True)) + m
    o_ref[...] = logit - lse


def _sc_agg_body(msg_hbm, idx_hbm, w_hbm, out_hbm, idx_all, w_all, rows, outbuf,
                 gsem0, gsem1, osem0, osem1):
    wid = lax.axis_index("s") * 2 + lax.axis_index("c")
    node0 = wid * NODES_PER_W
    row0 = wid * ROWS_PER_W

    pltpu.sync_copy(idx_hbm.at[pl.ds(row0, ROWS_PER_W)], idx_all)
    pltpu.sync_copy(w_hbm.at[pl.ds(row0, ROWS_PER_W)], w_all)

    gsems = (gsem0, gsem1)
    osems = (osem0, osem1)

    def fire(bc, b):
        for j in range(BC_GATHERS):
            pltpu.async_copy(msg_hbm.at[idx_all.at[bc * BC_GATHERS + j]],
                             rows.at[b, j], gsems[b])

    def drain(b):
        for j in range(BC_GATHERS):
            pltpu.make_async_copy(msg_hbm.at[idx_all.at[0]], rows.at[b, j],
                                  gsems[b]).wait()

    fire(0, 0)

    def pair_body(i, carry):
        for b in range(2):
            bc = i * 2 + b

            @pl.when(bc + 1 < N_BIG)
            def _():
                fire(bc + 1, 1 - b)

            drain(b)

            @pl.when(bc >= 2)
            def _():
                pltpu.make_async_copy(outbuf.at[b],
                                      out_hbm.at[pl.ds(0, BC_NODES)],
                                      osems[b]).wait()

            for n in range(BC_NODES):
                acc = [jnp.zeros((16,), jnp.float32) for _ in range(4)]
                for g in range(NB // 16):
                    epos = n * NB + g * 16
                    wvec = w_all[bc * BC_GATHERS + epos // 128,
                                 pl.ds(epos % 128, 16)]
                    for e in range(16):
                        j, r = (epos + e) // 128, (epos + e) % 128
                        wv = wvec[e]
                        for k in range(4):
                            acc[k] = acc[k] + wv * rows[b, j, r, pl.ds(k * 16, 16)]
                for k in range(4):
                    outbuf[b, n, pl.ds(k * 16, 16)] = jnp.maximum(acc[k], 0.0)

            pltpu.async_copy(outbuf.at[b],
                             out_hbm.at[pl.ds(node0 + bc * BC_NODES, BC_NODES)],
                             osems[b])
        return carry

    lax.fori_loop(0, N_BIG // 2, pair_body, 0)
    for b in range(2):
        pltpu.make_async_copy(outbuf.at[b], out_hbm.at[pl.ds(0, BC_NODES)],
                              osems[b]).wait()


def _make_sc_agg():
    return pl.kernel(
        _sc_agg_body,
        out_type=jax.ShapeDtypeStruct((NPAD, D), jnp.float32),
        mesh=plsc.VectorSubcoreMesh(core_axis_name="c", subcore_axis_name="s"),
        compiler_params=pltpu.CompilerParams(use_tc_tiling_on_sc=False),
        scratch_types=[
            pltpu.VMEM((ROWS_PER_W, 128), jnp.int32),
            pltpu.VMEM((ROWS_PER_W, 128), jnp.float32),
            pltpu.VMEM((2, BC_GATHERS, 128, D), jnp.float32),
            pltpu.VMEM((2, BC_NODES, D), jnp.float32),
            pltpu.SemaphoreType.DMA,
            pltpu.SemaphoreType.DMA,
            pltpu.SemaphoreType.DMA,
            pltpu.SemaphoreType.DMA,
        ],
    )


def _proj(f, W_feat, b_feat, W_gnn0):
    return pl.pallas_call(
        _proj_body,
        grid=(2,),
        in_specs=[
            pl.BlockSpec((N // 2, DIN), lambda i: (i, 0)),
            pl.BlockSpec((DIN, D), lambda i: (0, 0)),
            pl.BlockSpec((1, D), lambda i: (0, 0)),
            pl.BlockSpec((D, D), lambda i: (0, 0)),
        ],
        out_specs=pl.BlockSpec((N // 2, D), lambda i: (i, 0)),
        out_shape=jax.ShapeDtypeStruct((N, D), jnp.float32),
    )(f, W_feat, b_feat, W_gnn0)


def _mm(x, W):
    return pl.pallas_call(
        _mm_body,
        grid=(2,),
        in_specs=[
            pl.BlockSpec((NPAD // 2, D), lambda i: (i, 0)),
            pl.BlockSpec((D, D), lambda i: (0, 0)),
        ],
        out_specs=pl.BlockSpec((NPAD // 2, D), lambda i: (i, 0)),
        out_shape=jax.ShapeDtypeStruct((NPAD, D), jnp.float32),
    )(x, W)


def _head(x, cen, W_out, b_out):
    return pl.pallas_call(
        _head_body,
        grid=(2,),
        in_specs=[
            pl.BlockSpec((NPAD // 2, D), lambda i: (i, 0)),
            pl.BlockSpec((K, D), lambda i: (0, 0)),
            pl.BlockSpec((K, C), lambda i: (0, 0)),
            pl.BlockSpec((1, C), lambda i: (0, 0)),
        ],
        out_specs=pl.BlockSpec((NPAD // 2, C), lambda i: (i, 0)),
        out_shape=jax.ShapeDtypeStruct((NPAD, C), jnp.float32),
    )(x, cen, W_out, b_out)


def kernel(adj, weight, features, W_feat, b_feat, W_gnn0, W_gnn1, centroids,
           W_out, b_out):
    pad_rows = NW * ROWS_PER_W - N * NB // 128
    idx2d = jnp.pad(adj[0].astype(jnp.int32).reshape(-1, 128),
                    ((0, pad_rows), (0, 0)))
    w2d = jnp.pad(weight[0].astype(jnp.float32).reshape(-1, 128),
                  ((0, pad_rows), (0, 0)))

    sc_agg = _make_sc_agg()
    msg0 = _proj(features[0], W_feat, b_feat.reshape(1, D), W_gnn0)
    x1 = sc_agg(msg0, idx2d, w2d)
    msg1 = _mm(x1, W_gnn1)
    x2 = sc_agg(msg1, idx2d, w2d)
    out = _head(x2, centroids, W_out, b_out.reshape(1, C))
    return out[:N]


# speedup vs baseline: 6.3786x; 2.2458x over previous
import jax
import jax.numpy as jnp
from jax import lax
from jax.experimental import pallas as pl
from jax.experimental.pallas import tpu as pltpu
from jax.experimental.pallas import tpu_sc as plsc

N = 10000
NB = 32
DIN = 128
D = 64
K = 100
C = 40

NW = 32
NPAD = 10240
NODES_PER_W = NPAD // NW
ROWS_PER_W = NODES_PER_W * NB // 128
BC_NODES = 8
BC_GATHERS = BC_NODES * NB // 128
N_BIG = NODES_PER_W // BC_NODES

_PREC = jax.lax.Precision.HIGHEST
_DN = (((1,), (1,)), ((), ()))


def _proj_body(f_ref, wf_ref, bf_ref, wg_ref, o_ref):
    x = jnp.dot(f_ref[...], wf_ref[...], preferred_element_type=jnp.float32,
                precision=_PREC)
    x = jnp.maximum(x + bf_ref[...], 0.0)
    o_ref[...] = jnp.dot(x, wg_ref[...], preferred_element_type=jnp.float32,
                         precision=_PREC)


def _mm_body(x_ref, w_ref, o_ref):
    o_ref[...] = jnp.dot(x_ref[...], w_ref[...],
                         preferred_element_type=jnp.float32, precision=_PREC)


def _head_body(x_ref, c_ref, wo_ref, bo_ref, o_ref):
    x = x_ref[...]
    cen = c_ref[...]
    xsq = jnp.sum(x * x, axis=1, keepdims=True)
    csq = jnp.sum(cen * cen, axis=1)[None, :]
    cross = lax.dot_general(x, cen, _DN, precision=_PREC,
                            preferred_element_type=jnp.float32)
    sq = xsq + csq - 2.0 * cross
    sim = jnp.sqrt(jnp.maximum(sq, 1e-12))
    logit = jnp.dot(sim, wo_ref[...], preferred_element_type=jnp.float32,
                    precision=_PREC) + bo_ref[...]
    m = jnp.max(logit, axis=1, keepdims=True)
    lse = jnp.log(jnp.sum(jnp.exp(logit - m), axis=1, keepdims=True)) + m
    o_ref[...] = logit - lse


def _sc_agg_body(msg_hbm, idx_hbm, w_hbm, out_hbm, idx_all, w_all, rows, outbuf,
                 table, gsem0, gsem1, osem0, osem1, tsem):
    sid = lax.axis_index("s")
    wid = sid * 2 + lax.axis_index("c")
    node0 = wid * NODES_PER_W
    row0 = wid * ROWS_PER_W
    vtile = msg_hbm.shape[0] // 16

    pltpu.async_copy(msg_hbm.at[pl.ds(sid * vtile, vtile)],
                     table.at[pl.ds(sid * vtile, vtile)], tsem)
    pltpu.sync_copy(idx_hbm.at[pl.ds(row0, ROWS_PER_W)], idx_all)
    pltpu.sync_copy(w_hbm.at[pl.ds(row0, ROWS_PER_W)], w_all)
    pltpu.make_async_copy(msg_hbm.at[pl.ds(0, vtile)],
                          table.at[pl.ds(0, vtile)], tsem).wait()
    plsc.subcore_barrier()

    gsems = (gsem0, gsem1)
    osems = (osem0, osem1)

    def fire(bc, b):
        for j in range(BC_GATHERS):
            pltpu.async_copy(table.at[idx_all.at[bc * BC_GATHERS + j]],
                             rows.at[b, j], gsems[b])

    def drain(b):
        for j in range(BC_GATHERS):
            pltpu.make_async_copy(table.at[idx_all.at[0]], rows.at[b, j],
                                  gsems[b]).wait()

    fire(0, 0)

    def pair_body(i, carry):
        for b in range(2):
            bc = i * 2 + b

            @pl.when(bc + 1 < N_BIG)
            def _():
                fire(bc + 1, 1 - b)

            drain(b)

            @pl.when(bc >= 2)
            def _():
                pltpu.make_async_copy(outbuf.at[b],
                                      out_hbm.at[pl.ds(0, BC_NODES)],
                                      osems[b]).wait()

            for n in range(BC_NODES):
                acc = [jnp.zeros((16,), jnp.float32) for _ in range(4)]
                for g in range(NB // 16):
                    epos = n * NB + g * 16
                    wvec = w_all[bc * BC_GATHERS + epos // 128,
                                 pl.ds(epos % 128, 16)]
                    for e in range(16):
                        j, r = (epos + e) // 128, (epos + e) % 128
                        wv = wvec[e]
                        for k in range(4):
                            acc[k] = acc[k] + wv * rows[b, j, r, pl.ds(k * 16, 16)]
                for k in range(4):
                    outbuf[b, n, pl.ds(k * 16, 16)] = jnp.maximum(acc[k], 0.0)

            pltpu.async_copy(outbuf.at[b],
                             out_hbm.at[pl.ds(node0 + bc * BC_NODES, BC_NODES)],
                             osems[b])
        return carry

    lax.fori_loop(0, N_BIG // 2, pair_body, 0)
    for b in range(2):
        pltpu.make_async_copy(outbuf.at[b], out_hbm.at[pl.ds(0, BC_NODES)],
                              osems[b]).wait()


def _make_sc_agg(v_rows):
    return pl.kernel(
        _sc_agg_body,
        out_type=jax.ShapeDtypeStruct((NPAD, D), jnp.float32),
        mesh=plsc.VectorSubcoreMesh(core_axis_name="c", subcore_axis_name="s"),
        compiler_params=pltpu.CompilerParams(use_tc_tiling_on_sc=False),
        scratch_types=[
            pltpu.VMEM((ROWS_PER_W, 128), jnp.int32),
            pltpu.VMEM((ROWS_PER_W, 128), jnp.float32),
            pltpu.VMEM((2, BC_GATHERS, 128, D), jnp.float32),
            pltpu.VMEM((2, BC_NODES, D), jnp.float32),
            pltpu.VMEM_SHARED((v_rows, D), jnp.float32),
            pltpu.SemaphoreType.DMA,
            pltpu.SemaphoreType.DMA,
            pltpu.SemaphoreType.DMA,
            pltpu.SemaphoreType.DMA,
            pltpu.SemaphoreType.DMA,
        ],
    )


def _proj(f, W_feat, b_feat, W_gnn0):
    return pl.pallas_call(
        _proj_body,
        grid=(2,),
        in_specs=[
            pl.BlockSpec((N // 2, DIN), lambda i: (i, 0)),
            pl.BlockSpec((DIN, D), lambda i: (0, 0)),
            pl.BlockSpec((1, D), lambda i: (0, 0)),
            pl.BlockSpec((D, D), lambda i: (0, 0)),
        ],
        out_specs=pl.BlockSpec((N // 2, D), lambda i: (i, 0)),
        out_shape=jax.ShapeDtypeStruct((N, D), jnp.float32),
    )(f, W_feat, b_feat, W_gnn0)


def _mm(x, W):
    return pl.pallas_call(
        _mm_body,
        grid=(2,),
        in_specs=[
            pl.BlockSpec((NPAD // 2, D), lambda i: (i, 0)),
            pl.BlockSpec((D, D), lambda i: (0, 0)),
        ],
        out_specs=pl.BlockSpec((NPAD // 2, D), lambda i: (i, 0)),
        out_shape=jax.ShapeDtypeStruct((NPAD, D), jnp.float32),
    )(x, W)


def _head(x, cen, W_out, b_out):
    return pl.pallas_call(
        _head_body,
        grid=(2,),
        in_specs=[
            pl.BlockSpec((NPAD // 2, D), lambda i: (i, 0)),
            pl.BlockSpec((K, D), lambda i: (0, 0)),
            pl.BlockSpec((K, C), lambda i: (0, 0)),
            pl.BlockSpec((1, C), lambda i: (0, 0)),
        ],
        out_specs=pl.BlockSpec((NPAD // 2, C), lambda i: (i, 0)),
        out_shape=jax.ShapeDtypeStruct((NPAD, C), jnp.float32),
    )(x, cen, W_out, b_out)


def kernel(adj, weight, features, W_feat, b_feat, W_gnn0, W_gnn1, centroids,
           W_out, b_out):
    pad_rows = NW * ROWS_PER_W - N * NB // 128
    idx2d = jnp.pad(adj[0].astype(jnp.int32).reshape(-1, 128),
                    ((0, pad_rows), (0, 0)))
    w2d = jnp.pad(weight[0].astype(jnp.float32).reshape(-1, 128),
                  ((0, pad_rows), (0, 0)))

    msg0 = _proj(features[0], W_feat, b_feat.reshape(1, D), W_gnn0)
    x1 = _make_sc_agg(N)(msg0, idx2d, w2d)
    msg1 = _mm(x1, W_gnn1)
    x2 = _make_sc_agg(NPAD)(msg1, idx2d, w2d)
    out = _head(x2, centroids, W_out, b_out.reshape(1, C))
    return out[:N]


# speedup vs baseline: 7.3778x; 1.1566x over previous
import jax
import jax.numpy as jnp
from jax import lax
from jax.experimental import pallas as pl
from jax.experimental.pallas import tpu as pltpu
from jax.experimental.pallas import tpu_sc as plsc

N = 10000
NB = 32
DIN = 128
D = 64
K = 100
C = 40

NW = 32
NODES_PER_W = 320
BC_NODES = 8
BC_GATHERS = BC_NODES * NB // 128
N_BIG = NODES_PER_W // BC_NODES
LAST_W = N // NODES_PER_W
LAST_NODES = N - LAST_W * NODES_PER_W
LAST_BIG = LAST_NODES // BC_NODES

RG = 5
RB = N // RG

_DN = (((1,), (1,)), ((), ()))


def _proj_body(f_ref, wf_ref, bf_ref, wg_ref, o_ref):
    x = jnp.dot(f_ref[0], wf_ref[...], preferred_element_type=jnp.float32)
    x = jnp.maximum(x + bf_ref[...], 0.0)
    o_ref[...] = jnp.dot(x, wg_ref[...], preferred_element_type=jnp.float32)


def _mm_body(x_ref, w_ref, o_ref):
    o_ref[...] = jnp.dot(x_ref[...], w_ref[...],
                         preferred_element_type=jnp.float32)


def _head_body(x_ref, c_ref, wo_ref, bo_ref, o_ref):
    x = x_ref[...]
    cen = c_ref[...]
    xsq = jnp.sum(x * x, axis=1, keepdims=True)
    csq = jnp.sum(cen * cen, axis=1)[None, :]
    cross = lax.dot_general(x, cen, _DN,
                            preferred_element_type=jnp.float32)
    sq = xsq + csq - 2.0 * cross
    sim = jnp.sqrt(jnp.maximum(sq, 1e-12))
    logit = jnp.dot(sim, wo_ref[...],
                    preferred_element_type=jnp.float32) + bo_ref[...]
    m = jnp.max(logit, axis=1, keepdims=True)
    lse = jnp.log(jnp.sum(jnp.exp(logit - m), axis=1, keepdims=True)) + m
    o_ref[...] = logit - lse


def _sc_agg_body(msg_hbm, idx_hbm, w_hbm, out_hbm, idx_all, w_all, rows, outbuf,
                 table, gsem0, gsem1, osem0, osem1, tsem):
    sid = lax.axis_index("s")
    wid = sid * 2 + lax.axis_index("c")
    node0 = wid * NODES_PER_W
    vtile = N // 16

    pltpu.async_copy(msg_hbm.at[pl.ds(sid * vtile, vtile)],
                     table.at[pl.ds(sid * vtile, vtile)], tsem)

    n_big = jnp.where(wid == LAST_W, LAST_BIG, N_BIG)
    edge0 = node0 * NB

    @pl.when(wid < LAST_W)
    def _():
        pltpu.sync_copy(idx_hbm.at[pl.ds(edge0, NODES_PER_W * NB)], idx_all)
        pltpu.sync_copy(w_hbm.at[pl.ds(edge0, NODES_PER_W * NB)], w_all)

    @pl.when(wid == LAST_W)
    def _():
        pltpu.sync_copy(idx_hbm.at[pl.ds(edge0, LAST_NODES * NB)],
                        idx_all.at[pl.ds(0, LAST_NODES * NB)])
        pltpu.sync_copy(w_hbm.at[pl.ds(edge0, LAST_NODES * NB)],
                        w_all.at[pl.ds(0, LAST_NODES * NB)])

    pltpu.make_async_copy(msg_hbm.at[pl.ds(0, vtile)],
                          table.at[pl.ds(0, vtile)], tsem).wait()
    plsc.subcore_barrier()

    gsems = (gsem0, gsem1)
    osems = (osem0, osem1)

    def fire(bc, b):
        for j in range(BC_GATHERS):
            pltpu.async_copy(
                table.at[idx_all.at[pl.ds((bc * BC_GATHERS + j) * 128, 128)]],
                rows.at[b, j], gsems[b])

    def drain(b):
        for j in range(BC_GATHERS):
            pltpu.make_async_copy(table.at[idx_all.at[pl.ds(0, 128)]],
                                  rows.at[b, j], gsems[b]).wait()

    fire(0, 0)

    def pair_body(i, carry):
        for b in range(2):
            bc = i * 2 + b

            @pl.when(bc + 1 < n_big)
            def _():
                fire(bc + 1, 1 - b)

            drain(b)

            @pl.when(bc >= 2)
            def _():
                pltpu.make_async_copy(outbuf.at[b],
                                      out_hbm.at[pl.ds(0, BC_NODES)],
                                      osems[b]).wait()

            for n in range(BC_NODES):
                acc = [jnp.zeros((16,), jnp.float32) for _ in range(4)]
                for g in range(NB // 16):
                    wvec = w_all[pl.ds((bc * BC_NODES + n) * NB + g * 16, 16)]
                    for e in range(16):
                        epos = n * NB + g * 16 + e
                        j, r = epos // 128, epos % 128
                        wv = wvec[e]
                        for k in range(4):
                            acc[k] = acc[k] + wv * rows[b, j, r, pl.ds(k * 16, 16)]
                for k in range(4):
                    outbuf[b, n, pl.ds(k * 16, 16)] = jnp.maximum(acc[k], 0.0)

            pltpu.async_copy(outbuf.at[b],
                             out_hbm.at[pl.ds(node0 + bc * BC_NODES, BC_NODES)],
                             osems[b])
        return carry

    lax.fori_loop(0, n_big // 2, pair_body, 0)
    for b in range(2):
        pltpu.make_async_copy(outbuf.at[b], out_hbm.at[pl.ds(0, BC_NODES)],
                              osems[b]).wait()


def _make_sc_agg():
    return pl.kernel(
        _sc_agg_body,
        out_type=jax.ShapeDtypeStruct((N, D), jnp.float32),
        mesh=plsc.VectorSubcoreMesh(core_axis_name="c", subcore_axis_name="s"),
        compiler_params=pltpu.CompilerParams(use_tc_tiling_on_sc=False),
        scratch_types=[
            pltpu.VMEM((NODES_PER_W * NB,), jnp.int32),
            pltpu.VMEM((NODES_PER_W * NB,), jnp.float32),
            pltpu.VMEM((2, BC_GATHERS, 128, D), jnp.float32),
            pltpu.VMEM((2, BC_NODES, D), jnp.float32),
            pltpu.VMEM_SHARED((N, D), jnp.float32),
            pltpu.SemaphoreType.DMA,
            pltpu.SemaphoreType.DMA,
            pltpu.SemaphoreType.DMA,
            pltpu.SemaphoreType.DMA,
            pltpu.SemaphoreType.DMA,
        ],
    )


def _proj(f3, W_feat, b_feat, W_gnn0):
    return pl.pallas_call(
        _proj_body,
        grid=(RG,),
        in_specs=[
            pl.BlockSpec((1, RB, DIN), lambda i: (0, i, 0)),
            pl.BlockSpec((DIN, D), lambda i: (0, 0)),
            pl.BlockSpec((1, D), lambda i: (0, 0)),
            pl.BlockSpec((D, D), lambda i: (0, 0)),
        ],
        out_specs=pl.BlockSpec((RB, D), lambda i: (i, 0)),
        out_shape=jax.ShapeDtypeStruct((N, D), jnp.float32),
    )(f3, W_feat, b_feat, W_gnn0)


def _mm(x, W):
    return pl.pallas_call(
        _mm_body,
        grid=(RG,),
        in_specs=[
            pl.BlockSpec((RB, D), lambda i: (i, 0)),
            pl.BlockSpec((D, D), lambda i: (0, 0)),
        ],
        out_specs=pl.BlockSpec((RB, D), lambda i: (i, 0)),
        out_shape=jax.ShapeDtypeStruct((N, D), jnp.float32),
    )(x, W)


def _head(x, cen, W_out, b_out):
    return pl.pallas_call(
        _head_body,
        grid=(RG,),
        in_specs=[
            pl.BlockSpec((RB, D), lambda i: (i, 0)),
            pl.BlockSpec((K, D), lambda i: (0, 0)),
            pl.BlockSpec((K, C), lambda i: (0, 0)),
            pl.BlockSpec((1, C), lambda i: (0, 0)),
        ],
        out_specs=pl.BlockSpec((RB, C), lambda i: (i, 0)),
        out_shape=jax.ShapeDtypeStruct((N, C), jnp.float32),
    )(x, cen, W_out, b_out)


def kernel(adj, weight, features, W_feat, b_feat, W_gnn0, W_gnn1, centroids,
           W_out, b_out):
    idx = adj[0].astype(jnp.int32).reshape(N * NB)
    w = weight[0].astype(jnp.float32).reshape(N * NB)

    sc_agg = _make_sc_agg()
    msg0 = _proj(features, W_feat, b_feat.reshape(1, D), W_gnn0)
    x1 = sc_agg(msg0, idx, w)
    msg1 = _mm(x1, W_gnn1)
    x2 = sc_agg(msg1, idx, w)
    return _head(x2, centroids, W_out, b_out.reshape(1, C))
